# baseline (speedup 1.0000x reference)
import jax
import jax.numpy as jnp
from jax import lax
from jax.experimental import pallas as pl
from jax.experimental.pallas import tpu as pltpu
from jax.experimental.pallas import tpu_sc as plsc

_PQ, _PH, _PD = 192, 384, 768
_B, _T = 256, 196
_NROWS = _B * _T
_NC, _NS = 2, 16
_NW = _NC * _NS
_RPW = _NROWS // _NW
_R = 32
_CH = _RPW // _R
_NV = _PD // 16
_NIN, _NOUT = 3, 2


def _compute_rows(in_v, out_v, r0, r1):
  @plsc.parallel_loop(0, _NV, 1)
  def _v(v):
    p = lax.iota(jnp.int32, 16) + 16 * v
    m = p - _PQ
    src = jnp.where((p >= _PQ) & (p < _PH + _PQ),
                    _PQ + (m >> 1) + (m & 1) * _PQ, p)
    lo = 16 * v

    @plsc.parallel_loop(r0, r1, 1, unroll=8)
    def _row(r):
      rv = jnp.full((16,), r, jnp.int32)
      out_v[r, pl.ds(lo, 16)] = plsc.load_gather(in_v, [rv, src])


def _body(x_hbm, o_hbm, in0, in1, in2, out0, out1,
          si0, si1, si2, so0a, so0b, so1a, so1b):
  wid = lax.axis_index("s") * _NC + lax.axis_index("c")
  ins, outs = (in0, in1, in2), (out0, out1)
  sis = (si0, si1, si2)
  sos = ((so0a, so0b), (so1a, so1b))
  _H = _R // 2

  def _row0(c):
    return (c * _NW + wid) * _R

  def _wait_read(i):
    pltpu.make_async_copy(x_hbm.at[pl.ds(0, _R)], ins[i], sis[i]).wait()

  def _wait_write(i):
    for h in (0, 1):
      pltpu.make_async_copy(outs[i].at[pl.ds(0, _H)],
                            o_hbm.at[pl.ds(0, _H)], sos[i][h]).wait()

  def _start_read(c, i):
    pltpu.async_copy(x_hbm.at[pl.ds(_row0(c), _R)], ins[i], sis[i])

  def _start_write_half(c, i, h):
    pltpu.async_copy(outs[i].at[pl.ds(h * _H, _H)],
                     o_hbm.at[pl.ds(_row0(c) + h * _H, _H)], sos[i][h])

  for c in range(_NIN):
    _start_read(c, c)

  def group(g, carry):
    for b in range(6):
      c = 6 * g + b
      bi, bo = b % _NIN, b % _NOUT
      _wait_read(bi)

      @pl.when(c >= _NOUT)
      def _():
        _wait_write(bo)

      _compute_rows(ins[bi], outs[bo], 0, _H)
      _start_write_half(c, bo, 0)
      _compute_rows(ins[bi], outs[bo], _H, _R)
      _start_write_half(c, bo, 1)

      @pl.when(c + _NIN < _CH)
      def _():
        _start_read(c + _NIN, bi)

    return carry

  lax.fori_loop(0, _CH // 6, group, 0)

  c_last = _CH - 1
  _wait_read(0)
  _wait_write(0)
  _compute_rows(in0, out0, 0, _H)
  _start_write_half(c_last, 0, 0)
  _compute_rows(in0, out0, _H, _R)
  _start_write_half(c_last, 0, 1)

  _wait_write(0)
  _wait_write(1)


def kernel(inputs):
  x = inputs.transpose(1, 0, 2).reshape(_NROWS, _PD)
  mesh = plsc.VectorSubcoreMesh(
      core_axis_name="c", subcore_axis_name="s",
      num_cores=_NC, num_subcores=_NS)
  out = pl.kernel(
      _body,
      out_type=jax.ShapeDtypeStruct((_NROWS, _PD), jnp.float32),
      mesh=mesh,
      compiler_params=pltpu.CompilerParams(
          needs_layout_passes=False, disable_bounds_checks=True),
      scratch_types=[
          pltpu.VMEM((_R, _PD), jnp.float32),
          pltpu.VMEM((_R, _PD), jnp.float32),
          pltpu.VMEM((_R, _PD), jnp.float32),
          pltpu.VMEM((_R, _PD), jnp.float32),
          pltpu.VMEM((_R, _PD), jnp.float32),
          pltpu.SemaphoreType.DMA,
          pltpu.SemaphoreType.DMA,
          pltpu.SemaphoreType.DMA,
          pltpu.SemaphoreType.DMA,
          pltpu.SemaphoreType.DMA,
          pltpu.SemaphoreType.DMA,
          pltpu.SemaphoreType.DMA,
      ],
  )(x)
  return out.reshape(_T, _B, _PD).transpose(1, 0, 2)


# speedup vs baseline: 1.5774x; 1.5774x over previous
import jax
import jax.numpy as jnp
from jax import lax
from jax.experimental import pallas as pl
from jax.experimental.pallas import tpu as pltpu
from jax.experimental.pallas import tpu_sc as plsc

_PQ, _PH, _PD = 192, 384, 768
_B, _T = 256, 196
_NROWS = _B * _T
_NC, _NS = 2, 16
_NW = _NC * _NS
_RPW = _NROWS // _NW
_R = 32
_CH = _RPW // _R
_NV = _PD // 16
_NIN, _NOUT = 3, 2


def _compute_chunk(in_v, out_v):
  @plsc.parallel_loop(0, _NV, 1)
  def _v(v):
    p = lax.iota(jnp.int32, 16) + 16 * v
    m = p - _PQ
    src = jnp.where((p >= _PQ) & (p < _PH + _PQ),
                    _PQ + (m >> 1) + (m & 1) * _PQ, p)
    lo = 16 * v

    @plsc.parallel_loop(0, _R, 1, unroll=8)
    def _row(r):
      rv = jnp.full((16,), r, jnp.int32)
      out_v[r, pl.ds(lo, 16)] = plsc.load_gather(in_v, [rv, src])


def _body(x_hbm, o_hbm, in0, in1, in2, out0, out1, si0, si1, si2, so0, so1):
  wid = lax.axis_index("s") * _NC + lax.axis_index("c")
  ins, outs = (in0, in1, in2), (out0, out1)
  sis, sos = (si0, si1, si2), (so0, so1)

  def _row0(c):
    return (c * _NW + wid) * _R

  def _wait_read(i):
    pltpu.make_async_copy(x_hbm.at[pl.ds(0, _R)], ins[i], sis[i]).wait()

  def _wait_write(i):
    pltpu.make_async_copy(outs[i], o_hbm.at[pl.ds(0, _R)], sos[i]).wait()

  def _start_read(c, i):
    pltpu.async_copy(x_hbm.at[pl.ds(_row0(c), _R)], ins[i], sis[i])

  def _start_write(c, i):
    pltpu.async_copy(outs[i], o_hbm.at[pl.ds(_row0(c), _R)], sos[i])

  for c in range(_NIN):
    _start_read(c, c)

  def group(g, carry):
    for b in range(6):
      c = 6 * g + b
      bi, bo = b % _NIN, b % _NOUT
      _wait_read(bi)

      @pl.when(c >= _NOUT)
      def _():
        _wait_write(bo)

      _compute_chunk(ins[bi], outs[bo])

      @pl.when(c + _NIN < _CH)
      def _():
        _start_read(c + _NIN, bi)

      _start_write(c, bo)

    return carry

  lax.fori_loop(0, _CH // 6, group, 0)

  c_last = _CH - 1
  _wait_read(0)
  _wait_write(0)
  _compute_chunk(in0, out0)
  _start_write(c_last, 0)

  _wait_write(0)
  _wait_write(1)


def kernel(inputs):
  x = inputs.transpose(1, 0, 2).reshape(_NROWS, _PD)
  mesh = plsc.VectorSubcoreMesh(
      core_axis_name="c", subcore_axis_name="s",
      num_cores=_NC, num_subcores=_NS)
  out = pl.kernel(
      _body,
      out_type=jax.ShapeDtypeStruct((_NROWS, _PD), jnp.float32),
      mesh=mesh,
      compiler_params=pltpu.CompilerParams(
          needs_layout_passes=False, disable_bounds_checks=True),
      scratch_types=[
          pltpu.VMEM((_R, _PD), jnp.float32),
          pltpu.VMEM((_R, _PD), jnp.float32),
          pltpu.VMEM((_R, _PD), jnp.float32),
          pltpu.VMEM((_R, _PD), jnp.float32),
          pltpu.VMEM((_R, _PD), jnp.float32),
          pltpu.SemaphoreType.DMA,
          pltpu.SemaphoreType.DMA,
          pltpu.SemaphoreType.DMA,
          pltpu.SemaphoreType.DMA,
          pltpu.SemaphoreType.DMA,
      ],
  )(x)
  return out.reshape(_T, _B, _PD).transpose(1, 0, 2)
